# Initial kernel scaffold; baseline (speedup 1.0000x reference)
import jax
import jax.numpy as jnp
from jax.experimental import pallas as pl


def kernel(x, edge_index, edge_attr, batch_vec, eW1, eb1, W11, b11, W12, b12, g1, be1, eW2, eb2, W21, b21, W22, b22, g2, be2, Wh1, bh1, Wh2, bh2):
    raise NotImplementedError("write your pallas kernel here")


# speedup vs baseline: 2.2411x; 2.2411x over previous
import functools

import jax
import jax.numpy as jnp
from jax import lax
from jax.experimental import pallas as pl
from jax.experimental.pallas import tpu as pltpu
from jax.experimental.pallas import tpu_sc as plsc

N = 10000
E = 320000
D = 128
B = 64

NC = 2
NS = 16
NW = NC * NS
EPT = E // NW
CHUNK = 80
NCHUNK = EPT // CHUNK
NPAD = 10240
RPT = NPAD // NS
NSUB = D // 16


def _sc_edge_body(h_hbm, src_hbm, dst_hbm, attr_hbm, ewb_hbm, zeros_hbm,
                  out_hbm, src_v, dst_v, attr_v, rows_v, w_v, aggr_sh, sem):
    cid = lax.axis_index("c")
    sid = lax.axis_index("s")
    pltpu.sync_copy(zeros_hbm.at[pl.ds(sid * RPT, RPT)],
                    aggr_sh.at[pl.ds(sid * RPT, RPT)])
    pltpu.sync_copy(ewb_hbm, w_v)
    plsc.subcore_barrier()

    wv = [[w_v[k, pl.ds(c * 16, 16)] for c in range(NSUB)] for k in range(5)]
    base = (cid * NS + sid) * EPT

    def chunk_body(g, carry):
        off = base + g * CHUNK
        pltpu.sync_copy(src_hbm.at[pl.ds(off, CHUNK)], src_v)
        pltpu.sync_copy(dst_hbm.at[pl.ds(off, CHUNK)], dst_v)
        pltpu.sync_copy(attr_hbm.at[pl.ds(off * 4, CHUNK * 4)],
                        attr_v.at[pl.ds(0, CHUNK * 4)])
        pltpu.async_copy(h_hbm.at[src_v], rows_v, sem).wait()

        def edge_body(i, c2):
            av = attr_v[pl.ds(i * 4, 16)]
            a0 = av[0]
            a1 = av[1]
            a2 = av[2]
            a3 = av[3]
            for c in range(NSUB):
                sl = pl.ds(c * 16, 16)
                e = (a0 * wv[0][c] + a1 * wv[1][c] + a2 * wv[2][c]
                     + a3 * wv[3][c] + wv[4][c])
                rows_v[i, sl] = jnp.maximum(rows_v[i, sl] + e, 0.0)
            return c2

        lax.fori_loop(0, CHUNK, edge_body, 0)
        pltpu.sync_copy(rows_v, aggr_sh.at[dst_v], add=True)
        return carry

    lax.fori_loop(0, NCHUNK, chunk_body, 0)
    plsc.subcore_barrier()
    pltpu.sync_copy(aggr_sh.at[pl.ds(sid * RPT, RPT)],
                    out_hbm.at[cid, pl.ds(sid * RPT, RPT)])


@functools.cache
def _build_sc_edge_pass():
    return pl.kernel(
        _sc_edge_body,
        out_type=jax.ShapeDtypeStruct((NC, NPAD, D), jnp.float32),
        mesh=plsc.VectorSubcoreMesh(core_axis_name="c", subcore_axis_name="s",
                                    num_cores=NC, num_subcores=NS),
        scratch_types=[
            pltpu.VMEM((CHUNK,), jnp.int32),
            pltpu.VMEM((CHUNK,), jnp.int32),
            pltpu.VMEM((CHUNK * 4 + 16,), jnp.float32),
            pltpu.VMEM((CHUNK, D), jnp.float32),
            pltpu.VMEM((5, D), jnp.float32),
            pltpu.VMEM_SHARED((NPAD, D), jnp.float32),
            pltpu.SemaphoreType.DMA,
        ],
    )


def _sc_edge_pass(*args):
    return _build_sc_edge_pass()(*args)


RB = 1000
GRID = N // RB


def _tc_layer_body(h_ref, p_ref, W1_ref, b1_ref, W2_ref, b2_ref, g_ref,
                   be_ref, out_ref):
    z = h_ref[...] + p_ref[0] + p_ref[1]
    t = jnp.dot(z, W1_ref[...], preferred_element_type=jnp.float32,
                precision=lax.Precision.HIGHEST)
    t = jnp.maximum(t + b1_ref[...], 0.0)
    t = jnp.dot(t, W2_ref[...], preferred_element_type=jnp.float32,
                precision=lax.Precision.HIGHEST)
    t = t + b2_ref[...]
    mu = jnp.mean(t, axis=-1, keepdims=True)
    var = jnp.mean(jnp.square(t - mu), axis=-1, keepdims=True)
    t = (t - mu) * lax.rsqrt(var + 1e-5) * g_ref[...] + be_ref[...]
    out_ref[...] = jnp.maximum(t, 0.0)


def _tc_layer(h, p, W1, b1, W2, b2, g, be):
    full = lambda *s: pl.BlockSpec(s, lambda i: tuple(0 for _ in s))
    return pl.pallas_call(
        _tc_layer_body,
        grid=(GRID,),
        in_specs=[
            pl.BlockSpec((RB, D), lambda i: (i, 0)),
            pl.BlockSpec((NC, RB, D), lambda i: (0, i, 0)),
            full(D, D), full(D), full(D, D), full(D), full(D), full(D),
        ],
        out_specs=pl.BlockSpec((RB, D), lambda i: (i, 0)),
        out_shape=jax.ShapeDtypeStruct((N, D), jnp.float32),
    )(h, p, W1, b1, W2, b2, g, be)


def _tc_final_body(h_ref, p_ref, bv_ref, W1_ref, b1_ref, W2_ref, b2_ref,
                   g_ref, be_ref, Wh1_ref, bh1_ref, Wh2_ref, bh2_ref,
                   nl_ref, pooled_ref, cnt_ref, gl_ref):
    i = pl.program_id(0)
    z = h_ref[...] + p_ref[0] + p_ref[1]
    t = jnp.dot(z, W1_ref[...], preferred_element_type=jnp.float32,
                precision=lax.Precision.HIGHEST)
    t = jnp.maximum(t + b1_ref[...], 0.0)
    t = jnp.dot(t, W2_ref[...], preferred_element_type=jnp.float32,
                precision=lax.Precision.HIGHEST)
    t = t + b2_ref[...]
    mu = jnp.mean(t, axis=-1, keepdims=True)
    var = jnp.mean(jnp.square(t - mu), axis=-1, keepdims=True)
    t = (t - mu) * lax.rsqrt(var + 1e-5) * g_ref[...] + be_ref[...]
    h2 = jnp.maximum(t, 0.0)

    q = jnp.maximum(
        jnp.dot(h2, Wh1_ref[...], preferred_element_type=jnp.float32,
                precision=lax.Precision.HIGHEST)
        + bh1_ref[...], 0.0)
    nl_ref[...] = (jnp.dot(q, Wh2_ref[...], preferred_element_type=jnp.float32,
                precision=lax.Precision.HIGHEST)
                   + bh2_ref[...])

    bv = bv_ref[...][0, 0, :]
    onehot = (bv[:, None]
              == lax.broadcasted_iota(jnp.int32, (1, B), 1)).astype(jnp.float32)

    @pl.when(i == 0)
    def _():
        pooled_ref[...] = jnp.zeros_like(pooled_ref)
        cnt_ref[...] = jnp.zeros_like(cnt_ref)

    pooled_ref[...] += lax.dot_general(onehot, h2, (((0,), (0,)), ((), ())),
                    precision=lax.Precision.HIGHEST)
    cnt_ref[...] += jnp.broadcast_to(jnp.sum(onehot, axis=0, keepdims=True),
                                     cnt_ref.shape)

    @pl.when(i == pl.num_programs(0) - 1)
    def _():
        gh = pooled_ref[...] / jnp.maximum(cnt_ref[...], 1.0)[0][:, None]
        gq = jnp.maximum(
            jnp.dot(gh, Wh1_ref[...], preferred_element_type=jnp.float32,
                precision=lax.Precision.HIGHEST)
            + bh1_ref[...], 0.0)
        gl_ref[...] = (jnp.dot(gq, Wh2_ref[...],
                               preferred_element_type=jnp.float32,
                precision=lax.Precision.HIGHEST)
                       + bh2_ref[...])


def _tc_final(h, p, bv, W1, b1, W2, b2, g, be, Wh1, bh1, Wh2, bh2):
    full = lambda *s: pl.BlockSpec(s, lambda i: tuple(0 for _ in s))
    return pl.pallas_call(
        _tc_final_body,
        grid=(GRID,),
        in_specs=[
            pl.BlockSpec((RB, D), lambda i: (i, 0)),
            pl.BlockSpec((NC, RB, D), lambda i: (0, i, 0)),
            pl.BlockSpec((1, 1, RB), lambda i: (i, 0, 0)),
            full(D, D), full(D), full(D, D), full(D), full(D), full(D),
            full(D, D // 2), full(D // 2), full(D // 2, 1), full(1),
        ],
        out_specs=[
            pl.BlockSpec((RB, 1), lambda i: (i, 0)),
            pl.BlockSpec((B, D), lambda i: (0, 0)),
            pl.BlockSpec((8, B), lambda i: (0, 0)),
            pl.BlockSpec((B, 1), lambda i: (0, 0)),
        ],
        out_shape=[
            jax.ShapeDtypeStruct((N, 1), jnp.float32),
            jax.ShapeDtypeStruct((B, D), jnp.float32),
            jax.ShapeDtypeStruct((8, B), jnp.float32),
            jax.ShapeDtypeStruct((B, 1), jnp.float32),
        ],
    )(h, p, bv, W1, b1, W2, b2, g, be, Wh1, bh1, Wh2, bh2)


def kernel(x, edge_index, edge_attr, batch_vec, eW1, eb1, W11, b11, W12, b12,
           g1, be1, eW2, eb2, W21, b21, W22, b22, g2, be2, Wh1, bh1, Wh2, bh2):
    src = edge_index[0]
    dst = edge_index[1]
    edge_attr = edge_attr.reshape(E * 4)
    zeros = jnp.zeros((NPAD, D), jnp.float32)
    ewb1 = jnp.concatenate([eW1, eb1[None, :]], axis=0)
    ewb2 = jnp.concatenate([eW2, eb2[None, :]], axis=0)

    p1 = _sc_edge_pass(x, src, dst, edge_attr, ewb1, zeros)
    h1 = _tc_layer(x, p1, W11, b11, W12, b12, g1, be1)
    p2 = _sc_edge_pass(h1, src, dst, edge_attr, ewb2, zeros)
    batch_vec = batch_vec.reshape(GRID, 1, RB)
    node_logits, _, _, gl = _tc_final(h1, p2, batch_vec, W21, b21, W22, b22,
                                      g2, be2, Wh1, bh1, Wh2, bh2)
    return (node_logits, gl[:, 0])
